# baseline (speedup 1.0000x reference)
import functools

import jax
import jax.numpy as jnp
from jax import lax
from jax.experimental import pallas as pl
from jax.experimental.pallas import tpu as pltpu
from jax.experimental.pallas import tpu_sc as plsc

B, S = 16, 1500
CODES, SW, W = 1024, 768, 1024
NT = B * S
NWORKERS = 32
LANES = 16
TPT = 768
SPT = TPT // B
C = 32
NBUF = 3
KFULL = TPT // C
KLAST = (NT - 31 * TPT) // C
PWIN = SPT // 2


def _mm_body(a_ref, b_ref, bias_ref, sp_ref, o_ref):
    o_ref[pl.ds(0, CODES), :] = (
        jnp.dot(a_ref[...], b_ref[...], preferred_element_type=jnp.float32)
        + bias_ref[...]
    )
    o_ref[pl.ds(CODES, 8), :] = jnp.broadcast_to(sp_ref[...], (8, W))


def _project_table(main_w, e2h_w, e2h_b, special_w):
    return pl.pallas_call(
        _mm_body,
        out_shape=jax.ShapeDtypeStruct((CODES + 8, W), jnp.float32),
    )(main_w, e2h_w, e2h_b.reshape(1, W), special_w)


def _sc_body(table, idxs, pos_emb, out, idx_v, pos_v, rows_v, sem_g, sem_w):
    wid = lax.axis_index("s") * 2 + lax.axis_index("c")
    t0 = wid * TPT
    is_last = wid == NWORKERS - 1

    pltpu.sync_copy(idxs.at[pl.ds(t0, TPT)], idx_v)
    pltpu.sync_copy(pos_emb.at[pl.ds(wid * SPT, PWIN)], pos_v)

    def gather(k, p):
        pltpu.async_copy(table.at[idx_v.at[pl.ds(C * k, C)]], rows_v[p], sem_g[p])

    def wait_gather(k, p):
        pltpu.make_async_copy(table.at[idx_v.at[pl.ds(C * k, C)]],
                              rows_v[p], sem_g[p]).wait()

    def write(k, p):
        pltpu.async_copy(rows_v[p], out.at[pl.ds(t0 + C * k, C)], sem_w[p])

    def wait_write(k, p):
        pltpu.make_async_copy(rows_v[p], out.at[pl.ds(t0 + C * k, C)],
                              sem_w[p]).wait()

    def add_pos(k, p):
        s0 = (C * k // B) % PWIN
        segs = ((0, B, s0), (B, B, s0 + 1))

        def j_body(j, carry):
            off = LANES * j
            for start, ln, sl in segs:
                v = pos_v[sl, pl.ds(off, LANES)]
                for i in range(start, start + ln):
                    plsc.addupdate(rows_v[p].at[i, pl.ds(off, LANES)], v)
            return carry
        lax.fori_loop(0, W // LANES, j_body, 0, unroll=False)

    def chunk_tail(k, p):
        if C * k // B == PWIN:
            pltpu.sync_copy(pos_emb.at[pl.ds(wid * SPT + PWIN, PWIN)], pos_v)
        wait_gather(k, p)
        add_pos(k, p)

    gather(0, 0)
    for k in range(KFULL):
        p = k % NBUF
        if 1 <= k <= KLAST:
            write(k - 1, (k - 1) % NBUF)
        elif k >= 1:
            @pl.when(~is_last)
            def _(k=k):
                write(k - 1, (k - 1) % NBUF)
        if k + 1 < KFULL:
            def prefetch(k=k):
                if k + 1 - NBUF >= 0:
                    wait_write(k + 1 - NBUF, (k + 1) % NBUF)
                gather(k + 1, (k + 1) % NBUF)
            if k + 1 < KLAST:
                prefetch()
            else:
                pl.when(~is_last)(prefetch)
        if k < KLAST:
            chunk_tail(k, p)
        else:
            @pl.when(~is_last)
            def _():
                chunk_tail(k, p)
    @pl.when(~is_last)
    def _():
        write(KFULL - 1, (KFULL - 1) % NBUF)
        for k in range(KFULL - NBUF, KFULL):
            wait_write(k, k % NBUF)

    @pl.when(is_last)
    def _():
        for k in range(max(KLAST - NBUF, 0), KLAST):
            wait_write(k, k % NBUF)


@functools.partial(
    pl.kernel,
    out_type=jax.ShapeDtypeStruct((NT, W), jnp.float32),
    mesh=plsc.VectorSubcoreMesh(core_axis_name="c", subcore_axis_name="s"),
    scratch_types=[
        pltpu.VMEM((TPT,), jnp.int32),
        pltpu.VMEM((PWIN, W), jnp.float32),
        [pltpu.VMEM((C, W), jnp.float32)] * NBUF,
        [pltpu.SemaphoreType.DMA] * NBUF,
        [pltpu.SemaphoreType.DMA] * NBUF,
    ],
)
def _sc_gather_add(table, idxs, pos_emb, out, idx_v, pos_v, rows_v, sem_g, sem_w):
    _sc_body(table, idxs, pos_emb, out, idx_v, pos_v, rows_v, sem_g, sem_w)


def kernel(Stoks, xenc, main_w, special_w, e2h_w, e2h_b, pos_emb):
    table = _project_table(main_w, e2h_w, e2h_b, special_w)
    pos_pad = jnp.pad(pos_emb, ((0, NWORKERS * SPT - S), (0, 0)))
    idxs = jnp.transpose(Stoks).reshape(NT).astype(jnp.int32)
    out = _sc_gather_add(table, idxs, pos_pad)
    xin = jnp.transpose(out.reshape(S, B, W), (1, 0, 2))
    return (xin.astype(xenc.dtype), 0)


# speedup vs baseline: 1.0086x; 1.0086x over previous
import functools

import jax
import jax.numpy as jnp
from jax import lax
from jax.experimental import pallas as pl
from jax.experimental.pallas import tpu as pltpu
from jax.experimental.pallas import tpu_sc as plsc

B, S = 16, 1500
CODES, SW, W = 1024, 768, 1024
NT = B * S
NWORKERS = 32
LANES = 16
TPT = 768
SPT = TPT // B
C = 24
NBUF = 3
KFULL = TPT // C
KLAST = (NT - 31 * TPT) // C


def _mm_body(a_ref, b_ref, bias_ref, sp_ref, o_ref):
    o_ref[pl.ds(0, CODES), :] = (
        jnp.dot(a_ref[...], b_ref[...], preferred_element_type=jnp.float32)
        + bias_ref[...]
    )
    o_ref[pl.ds(CODES, 8), :] = jnp.broadcast_to(sp_ref[...], (8, W))


def _project_table(main_w, e2h_w, e2h_b, special_w):
    return pl.pallas_call(
        _mm_body,
        out_shape=jax.ShapeDtypeStruct((CODES + 8, W), jnp.float32),
    )(main_w, e2h_w, e2h_b.reshape(1, W), special_w)


def _sc_body(table, idxs, pos_emb, out, idx_v, pos_v, rows_v, sem_g, sem_w):
    wid = lax.axis_index("s") * 2 + lax.axis_index("c")
    t0 = wid * TPT
    is_last = wid == NWORKERS - 1

    pltpu.sync_copy(idxs.at[pl.ds(t0, TPT)], idx_v)
    pltpu.sync_copy(pos_emb.at[pl.ds(wid * SPT, SPT)], pos_v)

    def gather(k, p):
        pltpu.async_copy(table.at[idx_v.at[pl.ds(C * k, C)]], rows_v[p], sem_g[p])

    def wait_gather(k, p):
        pltpu.make_async_copy(table.at[idx_v.at[pl.ds(C * k, C)]],
                              rows_v[p], sem_g[p]).wait()

    def write(k, p):
        pltpu.async_copy(rows_v[p], out.at[pl.ds(t0 + C * k, C)], sem_w[p])

    def wait_write(k, p):
        pltpu.make_async_copy(rows_v[p], out.at[pl.ds(t0 + C * k, C)],
                              sem_w[p]).wait()

    def add_pos(k, p):
        m = (C * k) % B
        s0 = (C * k) // B
        if m == 0:
            segs = ((0, B, s0), (B, C - B, s0 + 1))
        else:
            segs = ((0, B - m, s0), (B - m, C - (B - m), s0 + 1))

        def j_body(j, carry):
            off = LANES * j
            for start, ln, sl in segs:
                v = pos_v[sl, pl.ds(off, LANES)]
                for i in range(start, start + ln):
                    plsc.addupdate(rows_v[p].at[i, pl.ds(off, LANES)], v)
            return carry
        lax.fori_loop(0, W // LANES, j_body, 0, unroll=False)

    def chunk_tail(k, p):
        wait_gather(k, p)
        add_pos(k, p)

    gather(0, 0)
    for k in range(KFULL):
        p = k % NBUF
        if 1 <= k <= KLAST:
            write(k - 1, (k - 1) % NBUF)
        elif k >= 1:
            @pl.when(~is_last)
            def _(k=k):
                write(k - 1, (k - 1) % NBUF)
        if k + 1 < KFULL:
            def prefetch(k=k):
                if k + 1 - NBUF >= 0:
                    wait_write(k + 1 - NBUF, (k + 1) % NBUF)
                gather(k + 1, (k + 1) % NBUF)
            if k + 1 < KLAST:
                prefetch()
            else:
                pl.when(~is_last)(prefetch)
        if k < KLAST:
            chunk_tail(k, p)
        else:
            @pl.when(~is_last)
            def _():
                chunk_tail(k, p)
    @pl.when(~is_last)
    def _():
        write(KFULL - 1, (KFULL - 1) % NBUF)
        for k in range(KFULL - NBUF, KFULL):
            wait_write(k, k % NBUF)

    @pl.when(is_last)
    def _():
        for k in range(max(KLAST - NBUF, 0), KLAST):
            wait_write(k, k % NBUF)


@functools.partial(
    pl.kernel,
    out_type=jax.ShapeDtypeStruct((NT, W), jnp.float32),
    mesh=plsc.VectorSubcoreMesh(core_axis_name="c", subcore_axis_name="s"),
    scratch_types=[
        pltpu.VMEM((TPT,), jnp.int32),
        pltpu.VMEM((SPT, W), jnp.float32),
        [pltpu.VMEM((C, W), jnp.float32)] * NBUF,
        [pltpu.SemaphoreType.DMA] * NBUF,
        [pltpu.SemaphoreType.DMA] * NBUF,
    ],
)
def _sc_gather_add(table, idxs, pos_emb, out, idx_v, pos_v, rows_v, sem_g, sem_w):
    _sc_body(table, idxs, pos_emb, out, idx_v, pos_v, rows_v, sem_g, sem_w)


def kernel(Stoks, xenc, main_w, special_w, e2h_w, e2h_b, pos_emb):
    table = _project_table(main_w, e2h_w, e2h_b, special_w)
    pos_pad = jnp.pad(pos_emb, ((0, NWORKERS * SPT - S), (0, 0)))
    idxs = jnp.transpose(Stoks).reshape(NT).astype(jnp.int32)
    out = _sc_gather_add(table, idxs, pos_pad)
    xin = jnp.transpose(out.reshape(S, B, W), (1, 0, 2))
    return (xin.astype(xenc.dtype), 0)
